# baseline (speedup 1.0000x reference)
import jax
import jax.numpy as jnp
from jax import lax
from jax.experimental import pallas as pl
from jax.experimental.pallas import tpu as pltpu
from jax.experimental.pallas import tpu_sc as plsc

_DIM = 64
_SCALE = 8.0
_CHUNK = 128
_NW = 32


def _sc_embed(x2d, table):
    n_chunks = x2d.shape[0]
    n_rows = n_chunks * _CHUNK
    chunks_per_w = n_chunks // _NW

    mesh = plsc.VectorSubcoreMesh(core_axis_name="c", subcore_axis_name="s")

    @pl.kernel(
        out_type=jax.ShapeDtypeStruct((n_rows, _DIM), jnp.float32),
        mesh=mesh,
        scratch_types=[
            pltpu.VMEM((chunks_per_w, _CHUNK), jnp.int32),
            pltpu.VMEM((_CHUNK, _DIM), jnp.float32),
            pltpu.SemaphoreType.DMA,
        ],
        compiler_params=pltpu.CompilerParams(use_tc_tiling_on_sc=False),
    )
    def k(x_hbm, table_hbm, out_hbm, idx_v, rows_v, gsem):
        wid = lax.axis_index("s") * 2 + lax.axis_index("c")
        crow0 = wid * chunks_per_w
        pltpu.sync_copy(x_hbm.at[pl.ds(crow0, chunks_per_w)], idx_v)

        def chunk_body(j, carry):
            pltpu.async_copy(table_hbm.at[idx_v.at[j]], rows_v, gsem).wait()

            def scale_row(r, c2):
                for kk in range(_DIM // 16):
                    sl = pl.ds(kk * 16, 16)
                    rows_v[r, sl] = rows_v[r, sl] * _SCALE
                return c2

            lax.fori_loop(0, _CHUNK, scale_row, 0, unroll=2)
            pltpu.sync_copy(
                rows_v, out_hbm.at[pl.ds((crow0 + j) * _CHUNK, _CHUNK)]
            )
            return carry

        lax.fori_loop(0, chunks_per_w, chunk_body, 0)

    return k(x2d, table)


def kernel(x, table):
    b, s = x.shape
    x2d = x.reshape(-1, _CHUNK).astype(jnp.int32)
    out = _sc_embed(x2d, table)
    return out.reshape(b, s, _DIM)
